# baseline (speedup 1.0000x reference)
import jax
import jax.numpy as jnp
from jax import lax
from jax.experimental import pallas as pl
from jax.experimental.pallas import tpu as pltpu
from jax.experimental.pallas import tpu_sc as plsc

NUM_SEGMENTS = 1024
ROWS = 100000
FEATS = 128
BLK = 625
NBLK = ROWS // BLK
NC = 2
NS = 16
COLS = FEATS // NC
SEG_PER_TILE = NUM_SEGMENTS // NS
BLK_PER_TILE = NBLK // NS
NBUF = 2
AHEAD = 1


def _seg_sum_body(h_hbm, ids_hbm, out_hbm, acc, idx2, obuf, *rest):
    dbufs = rest[:NBUF]
    sems = rest[NBUF:]
    cid = lax.axis_index("c")
    sid = lax.axis_index("s")
    c0 = cid * COLS

    zeros = jnp.zeros((16,), jnp.float32)
    for i in range(SEG_PER_TILE):
        for c in range(COLS // 16):
            obuf[i, pl.ds(c * 16, 16)] = zeros
    pltpu.sync_copy(obuf, acc.at[pl.ds(sid * SEG_PER_TILE, SEG_PER_TILE)])
    plsc.subcore_barrier()

    lo = sid * BLK_PER_TILE

    pltpu.sync_copy(ids_hbm.at[pl.ds(lo, BLK_PER_TILE)], idx2)

    def gather(k):
        return pltpu.async_copy(
            h_hbm.at[pl.ds((lo + k) * BLK, BLK), pl.ds(c0, COLS)],
            dbufs[k % NBUF],
            sems[k % NBUF],
        )

    def scatter(k):
        return pltpu.async_copy(
            dbufs[k % NBUF], acc.at[idx2.at[k]], sems[k % NBUF], add=True
        )

    g = {k: gather(k) for k in range(AHEAD)}
    s = {}
    for k in range(BLK_PER_TILE):
        g[k].wait()
        s[k] = scatter(k)
        nk = k + AHEAD
        if nk < BLK_PER_TILE:
            if k - AHEAD >= 0:
                s[k - AHEAD].wait()
            g[nk] = gather(nk)
    for k in range(BLK_PER_TILE - 2 * AHEAD, BLK_PER_TILE):
        s[k].wait()
    plsc.subcore_barrier()

    pltpu.sync_copy(acc.at[pl.ds(sid * SEG_PER_TILE, SEG_PER_TILE)], obuf)
    pltpu.sync_copy(
        obuf, out_hbm.at[pl.ds(sid * SEG_PER_TILE, SEG_PER_TILE), pl.ds(c0, COLS)]
    )


@jax.jit
def kernel(h, segment_ids):
    ids2d = segment_ids.reshape(NBLK, BLK).astype(jnp.int32)
    mesh = plsc.VectorSubcoreMesh(core_axis_name="c", subcore_axis_name="s")
    f = pl.kernel(
        _seg_sum_body,
        mesh=mesh,
        out_type=jax.ShapeDtypeStruct((NUM_SEGMENTS, FEATS), jnp.float32),
        scratch_types=(
            [
                pltpu.VMEM_SHARED((NUM_SEGMENTS, COLS), jnp.float32),
                pltpu.VMEM((BLK_PER_TILE, BLK), jnp.int32),
                pltpu.VMEM((SEG_PER_TILE, COLS), jnp.float32),
            ]
            + [pltpu.VMEM((BLK, COLS), jnp.float32) for _ in range(NBUF)]
            + [pltpu.SemaphoreType.DMA for _ in range(NBUF)]
        ),
        compiler_params=pltpu.CompilerParams(
            use_tc_tiling_on_sc=False, skip_device_barrier=True
        ),
    )
    return f(h, ids2d)


# speedup vs baseline: 1.0754x; 1.0754x over previous
import jax
import jax.numpy as jnp
from jax import lax
from jax.experimental import pallas as pl
from jax.experimental.pallas import tpu as pltpu
from jax.experimental.pallas import tpu_sc as plsc

NUM_SEGMENTS = 1024
ROWS = 100000
FEATS = 128
BLK = 250
NBLK = ROWS // BLK
NC = 2
NS = 16
COLS = FEATS // NC
SEG_PER_TILE = NUM_SEGMENTS // NS
BLK_PER_TILE = NBLK // NS
NBUF = 7
AHEAD = 3


def _seg_sum_body(h_hbm, ids_hbm, out_hbm, acc, idx2, obuf, isem, *rest):
    dbufs = rest[:NBUF]
    sems = rest[NBUF:]
    cid = lax.axis_index("c")
    sid = lax.axis_index("s")
    c0 = cid * COLS

    lo = sid * BLK_PER_TILE

    def gather(k):
        return pltpu.async_copy(
            h_hbm.at[pl.ds((lo + k) * BLK, BLK), pl.ds(c0, COLS)],
            dbufs[k % NBUF],
            sems[k % NBUF],
        )

    def scatter(k):
        return pltpu.async_copy(
            dbufs[k % NBUF], acc.at[idx2.at[k]], sems[k % NBUF], add=True
        )

    icopy = pltpu.async_copy(ids_hbm.at[pl.ds(lo, BLK_PER_TILE)], idx2, isem)
    g = {k: gather(k) for k in range(AHEAD)}

    zeros = jnp.zeros((16,), jnp.float32)
    for i in range(SEG_PER_TILE):
        for c in range(COLS // 16):
            obuf[i, pl.ds(c * 16, 16)] = zeros
    pltpu.sync_copy(obuf, acc.at[pl.ds(sid * SEG_PER_TILE, SEG_PER_TILE)])
    plsc.subcore_barrier()
    icopy.wait()

    s = {}
    for k in range(BLK_PER_TILE):
        g[k].wait()
        s[k] = scatter(k)
        nk = k + AHEAD
        if nk < BLK_PER_TILE:
            fk = nk - NBUF
            if fk >= 0:
                s[fk].wait()
            g[nk] = gather(nk)
    for k in range(max(0, BLK_PER_TILE - NBUF), BLK_PER_TILE):
        s[k].wait()
    plsc.subcore_barrier()

    pltpu.sync_copy(acc.at[pl.ds(sid * SEG_PER_TILE, SEG_PER_TILE)], obuf)
    pltpu.sync_copy(
        obuf, out_hbm.at[pl.ds(sid * SEG_PER_TILE, SEG_PER_TILE), pl.ds(c0, COLS)]
    )


@jax.jit
def kernel(h, segment_ids):
    ids2d = segment_ids.reshape(NBLK, BLK).astype(jnp.int32)
    mesh = plsc.VectorSubcoreMesh(core_axis_name="c", subcore_axis_name="s")
    f = pl.kernel(
        _seg_sum_body,
        mesh=mesh,
        out_type=jax.ShapeDtypeStruct((NUM_SEGMENTS, FEATS), jnp.float32),
        scratch_types=(
            [
                pltpu.VMEM_SHARED((NUM_SEGMENTS, COLS), jnp.float32),
                pltpu.VMEM((BLK_PER_TILE, BLK), jnp.int32),
                pltpu.VMEM((SEG_PER_TILE, COLS), jnp.float32),
                pltpu.SemaphoreType.DMA,
            ]
            + [pltpu.VMEM((BLK, COLS), jnp.float32) for _ in range(NBUF)]
            + [pltpu.SemaphoreType.DMA for _ in range(NBUF)]
        ),
        compiler_params=pltpu.CompilerParams(
            use_tc_tiling_on_sc=False, skip_device_barrier=True
        ),
    )
    return f(h, ids2d)


# speedup vs baseline: 1.1131x; 1.0350x over previous
import jax
import jax.numpy as jnp
from jax import lax
from jax.experimental import pallas as pl
from jax.experimental.pallas import tpu as pltpu
from jax.experimental.pallas import tpu_sc as plsc

NUM_SEGMENTS = 1024
ROWS = 100000
FEATS = 128
BLK = 250
NBLK = ROWS // BLK
NC = 2
NS = 16
COLS = FEATS // NC
SEG_PER_TILE = NUM_SEGMENTS // NS
BLK_PER_TILE = NBLK // NS
NBUF = 7
AHEAD = 2


def _seg_sum_body(h_hbm, ids_hbm, out_hbm, acc, idx2, obuf, isem, *rest):
    dbufs = rest[:NBUF]
    sems = rest[NBUF:]
    cid = lax.axis_index("c")
    sid = lax.axis_index("s")
    c0 = cid * COLS

    lo = sid * BLK_PER_TILE

    def gather(k):
        return pltpu.async_copy(
            h_hbm.at[pl.ds((lo + k) * BLK, BLK), pl.ds(c0, COLS)],
            dbufs[k % NBUF],
            sems[k % NBUF],
        )

    def scatter(k):
        return pltpu.async_copy(
            dbufs[k % NBUF], acc.at[idx2.at[k]], sems[k % NBUF], add=True
        )

    icopy = pltpu.async_copy(ids_hbm.at[pl.ds(lo, BLK_PER_TILE)], idx2, isem)
    g = {k: gather(k) for k in range(AHEAD)}

    zeros = jnp.zeros((16,), jnp.float32)
    for i in range(SEG_PER_TILE):
        for c in range(COLS // 16):
            obuf[i, pl.ds(c * 16, 16)] = zeros
    pltpu.sync_copy(obuf, acc.at[pl.ds(sid * SEG_PER_TILE, SEG_PER_TILE)])
    plsc.subcore_barrier()
    icopy.wait()

    s = {}
    for k in range(BLK_PER_TILE):
        g[k].wait()
        s[k] = scatter(k)
        nk = k + AHEAD
        if nk < BLK_PER_TILE:
            fk = nk - NBUF
            if fk >= 0:
                s[fk].wait()
            g[nk] = gather(nk)
    for k in range(max(0, BLK_PER_TILE - NBUF), BLK_PER_TILE):
        s[k].wait()
    plsc.subcore_barrier()

    pltpu.sync_copy(acc.at[pl.ds(sid * SEG_PER_TILE, SEG_PER_TILE)], obuf)
    pltpu.sync_copy(
        obuf, out_hbm.at[pl.ds(sid * SEG_PER_TILE, SEG_PER_TILE), pl.ds(c0, COLS)]
    )


@jax.jit
def kernel(h, segment_ids):
    ids2d = segment_ids.reshape(NBLK, BLK).astype(jnp.int32)
    mesh = plsc.VectorSubcoreMesh(core_axis_name="c", subcore_axis_name="s")
    f = pl.kernel(
        _seg_sum_body,
        mesh=mesh,
        out_type=jax.ShapeDtypeStruct((NUM_SEGMENTS, FEATS), jnp.float32),
        scratch_types=(
            [
                pltpu.VMEM_SHARED((NUM_SEGMENTS, COLS), jnp.float32),
                pltpu.VMEM((BLK_PER_TILE, BLK), jnp.int32),
                pltpu.VMEM((SEG_PER_TILE, COLS), jnp.float32),
                pltpu.SemaphoreType.DMA,
            ]
            + [pltpu.VMEM((BLK, COLS), jnp.float32) for _ in range(NBUF)]
            + [pltpu.SemaphoreType.DMA for _ in range(NBUF)]
        ),
        compiler_params=pltpu.CompilerParams(
            use_tc_tiling_on_sc=False, skip_device_barrier=True
        ),
    )
    return f(h, ids2d)
